# baseline (speedup 1.0000x reference)
import jax
import jax.numpy as jnp
from jax import lax
from jax.experimental import pallas as pl
from jax.experimental.pallas import tpu as pltpu
from jax.experimental.pallas import tpu_sc as plsc

NC = 2
NS = 16
L = 16
NW = NC * NS

BATCH = 16384
EMB = 64
B_W = BATCH // NW
CHUNK = 128
NCH = B_W // CHUNK
NBLK = B_W // L


def _rsqrt16(x):
    i = plsc.bitcast(x, jnp.int32)
    i = jnp.int32(0x5F3759DF) - (i >> 1)
    y = plsc.bitcast(i, jnp.float32)
    half = x * jnp.float32(0.5)
    for _ in range(3):
        y = y * (jnp.float32(1.5) - half * y * y)
    return y


def _tec_body(ph_h, pt_h, pr_h, nh_h, nt_h, nr_h, ent_h, rel_h,
              pos_out, neg_out,
              ih_v, it_v, ir_v, h_rows, t_rows, r_rows, res_v, sem):
    wid = lax.axis_index("s") * NC + lax.axis_index("c")
    base_row = wid * NCH

    iota = lax.iota(jnp.int32, L)

    def run_phase(idx_h_hbm, idx_t_hbm, idx_r_hbm, out_hbm):
        pltpu.sync_copy(idx_h_hbm.at[pl.ds(base_row, NCH)], ih_v)
        pltpu.sync_copy(idx_t_hbm.at[pl.ds(base_row, NCH)], it_v)
        pltpu.sync_copy(idx_r_hbm.at[pl.ds(base_row, NCH)], ir_v)
        descs = []
        for j in range(NCH):
            descs.append(pltpu.async_copy(
                ent_h.at[ih_v.at[j]], h_rows.at[pl.ds(j * CHUNK, CHUNK)], sem))
            descs.append(pltpu.async_copy(
                ent_h.at[it_v.at[j]], t_rows.at[pl.ds(j * CHUNK, CHUNK)], sem))
            descs.append(pltpu.async_copy(
                rel_h.at[ir_v.at[j]], r_rows.at[pl.ds(j * CHUNK, CHUNK)], sem))
        for d in descs:
            d.wait()

        def block(b, _):
            row0 = b * L + iota

            def norm_body(d, carry):
                ah, at = carry
                dv = jnp.full((L,), d, dtype=jnp.int32)
                hc = plsc.load_gather(h_rows, [row0, dv])
                tc = plsc.load_gather(t_rows, [row0, dv])
                return ah + hc * hc, at + tc * tc

            zero = jnp.zeros((L,), jnp.float32)
            nh2, nt2 = lax.fori_loop(0, EMB, norm_body, (zero, zero))
            invh = _rsqrt16(nh2)
            invt = _rsqrt16(nt2)

            def dist_body(d, acc):
                dv = jnp.full((L,), d, dtype=jnp.int32)
                hc = plsc.load_gather(h_rows, [row0, dv])
                tc = plsc.load_gather(t_rows, [row0, dv])
                rc = plsc.load_gather(r_rows, [row0, dv])
                return acc + jnp.abs(hc * invh + rc - tc * invt)

            acc = lax.fori_loop(0, EMB, dist_body, zero)
            res_v[pl.ds(b * L, L)] = acc
            return 0

        lax.fori_loop(0, NBLK, block, 0)
        pltpu.sync_copy(res_v, out_hbm.at[pl.ds(wid * B_W, B_W)])

    run_phase(ph_h, pt_h, pr_h, pos_out)
    run_phase(nh_h, nt_h, nr_h, neg_out)


@jax.jit
def kernel(pos_h, pos_t, pos_r, neg_h, neg_t, neg_r, ent_emb, rel_emb):
    mesh = plsc.VectorSubcoreMesh(core_axis_name="c", subcore_axis_name="s",
                                  num_cores=NC, num_subcores=NS)
    out_type = (jax.ShapeDtypeStruct((BATCH,), jnp.float32),
                jax.ShapeDtypeStruct((BATCH,), jnp.float32))
    scratch = [
        pltpu.VMEM((NCH, CHUNK), jnp.int32),
        pltpu.VMEM((NCH, CHUNK), jnp.int32),
        pltpu.VMEM((NCH, CHUNK), jnp.int32),
        pltpu.VMEM((B_W, EMB), jnp.float32),
        pltpu.VMEM((B_W, EMB), jnp.float32),
        pltpu.VMEM((B_W, EMB), jnp.float32),
        pltpu.VMEM((B_W,), jnp.float32),
        pltpu.SemaphoreType.DMA,
    ]
    run = pl.kernel(_tec_body, out_type=out_type, mesh=mesh,
                    scratch_types=scratch,
                    compiler_params=pltpu.CompilerParams(
                        use_tc_tiling_on_sc=False,
                        needs_layout_passes=False))
    r2 = lambda a: a.reshape(NW * NCH, CHUNK)
    return run(r2(pos_h), r2(pos_t), r2(pos_r),
               r2(neg_h), r2(neg_t), r2(neg_r), ent_emb, rel_emb)


# speedup vs baseline: 1.0300x; 1.0300x over previous
import jax
import jax.numpy as jnp
from jax import lax
from jax.experimental import pallas as pl
from jax.experimental.pallas import tpu as pltpu
from jax.experimental.pallas import tpu_sc as plsc

NC = 2
NS = 16
L = 16
NW = NC * NS

BATCH = 16384
EMB = 64
B_W = BATCH // NW
CHUNK = 128
NCH = B_W // CHUNK
NBLK = CHUNK // L
DG = 16


def _rsqrt16(x):
    i = plsc.bitcast(x, jnp.int32)
    i = jnp.int32(0x5F3759DF) - (i >> 1)
    y = plsc.bitcast(i, jnp.float32)
    half = x * jnp.float32(0.5)
    for _ in range(3):
        y = y * (jnp.float32(1.5) - half * y * y)
    return y


def _tec_body(ph_h, pt_h, pr_h, nh_h, nt_h, nr_h, ent_h, rel_h,
              pos_out, neg_out,
              ih_v, it_v, ir_v, ihg_v, itg_v, irg_v,
              hb0, hb1, tb0, tb1, rb0, rb1, res_v, sem):
    wid = lax.axis_index("s") * NC + lax.axis_index("c")
    iota = lax.iota(jnp.int32, L)
    hbufs, tbufs, rbufs = (hb0, hb1), (tb0, tb1), (rb0, rb1)

    def run_phase(idx_h_hbm, idx_t_hbm, idx_r_hbm, out_hbm):
        pltpu.sync_copy(idx_h_hbm.at[wid], ih_v)
        pltpu.sync_copy(idx_t_hbm.at[wid], it_v)
        pltpu.sync_copy(idx_r_hbm.at[wid], ir_v)
        for raw, shifted in ((ih_v, ihg_v), (it_v, itg_v), (ir_v, irg_v)):
            for r in range(NCH):
                for k in range(CHUNK // L):
                    sl = pl.ds(k * L, L)
                    shifted[r, sl] = raw[r, sl] >> 1

        def fire(c):
            buf = c & 1
            return (
                pltpu.async_copy(ent_h.at[ihg_v.at[c]], hbufs[buf], sem),
                pltpu.async_copy(ent_h.at[itg_v.at[c]], tbufs[buf], sem),
                pltpu.async_copy(rel_h.at[irg_v.at[c]], rbufs[buf], sem),
            )

        def compute(c):
            buf = c & 1
            hbuf, tbuf, rbuf = hbufs[buf], tbufs[buf], rbufs[buf]

            def block(b, _):
                row16 = b * L + iota
                eh = plsc.load_gather(ih_v, [jnp.full((L,), c, jnp.int32),
                                             row16])
                et = plsc.load_gather(it_v, [jnp.full((L,), c, jnp.int32),
                                             row16])
                er = plsc.load_gather(ir_v, [jnp.full((L,), c, jnp.int32),
                                             row16])
                hcol = (eh & 1) * EMB
                tcol = (et & 1) * EMB
                rcol = (er & 1) * EMB

                zero = jnp.zeros((L,), jnp.float32)

                def norm_body(g, carry):
                    ah, at = carry
                    d0 = g * DG
                    for dd in range(DG):
                        hc = plsc.load_gather(hbuf, [row16, hcol + (d0 + dd)])
                        tc = plsc.load_gather(tbuf, [row16, tcol + (d0 + dd)])
                        ah = ah + hc * hc
                        at = at + tc * tc
                    return ah, at

                nh2, nt2 = lax.fori_loop(0, EMB // DG, norm_body, (zero, zero))
                invh = _rsqrt16(nh2)
                invt = _rsqrt16(nt2)

                def dist_body(g, acc):
                    d0 = g * DG
                    for dd in range(DG):
                        hc = plsc.load_gather(hbuf, [row16, hcol + (d0 + dd)])
                        tc = plsc.load_gather(tbuf, [row16, tcol + (d0 + dd)])
                        rc = plsc.load_gather(rbuf, [row16, rcol + (d0 + dd)])
                        acc = acc + jnp.abs(hc * invh + rc - tc * invt)
                    return acc

                acc = lax.fori_loop(0, EMB // DG, dist_body, zero)
                res_v[pl.ds(c * CHUNK + b * L, L)] = acc
                return 0

            lax.fori_loop(0, NBLK, block, 0)

        descs = fire(0)
        for c in range(NCH):
            for d in descs:
                d.wait()
            if c + 1 < NCH:
                descs = fire(c + 1)
            compute(c)

        pltpu.sync_copy(res_v, out_hbm.at[pl.ds(wid * B_W, B_W)])

    run_phase(ph_h, pt_h, pr_h, pos_out)
    run_phase(nh_h, nt_h, nr_h, neg_out)


@jax.jit
def kernel(pos_h, pos_t, pos_r, neg_h, neg_t, neg_r, ent_emb, rel_emb):
    mesh = plsc.VectorSubcoreMesh(core_axis_name="c", subcore_axis_name="s",
                                  num_cores=NC, num_subcores=NS)
    out_type = (jax.ShapeDtypeStruct((BATCH,), jnp.float32),
                jax.ShapeDtypeStruct((BATCH,), jnp.float32))
    scratch = (
        [pltpu.VMEM((NCH, CHUNK), jnp.int32)] * 6 +
        [pltpu.VMEM((CHUNK, 2 * EMB), jnp.float32)] * 6 +
        [pltpu.VMEM((B_W,), jnp.float32),
         pltpu.SemaphoreType.DMA]
    )
    run = pl.kernel(_tec_body, out_type=out_type, mesh=mesh,
                    scratch_types=scratch,
                    compiler_params=pltpu.CompilerParams(
                        needs_layout_passes=False))
    r3 = lambda a: a.reshape(NW, NCH, CHUNK)
    return run(r3(pos_h), r3(pos_t), r3(pos_r),
               r3(neg_h), r3(neg_t), r3(neg_r),
               ent_emb.reshape(-1, 2 * EMB), rel_emb.reshape(-1, 2 * EMB))


# speedup vs baseline: 1.5132x; 1.4692x over previous
import jax
import jax.numpy as jnp
from jax import lax
from jax.experimental import pallas as pl
from jax.experimental.pallas import tpu as pltpu
from jax.experimental.pallas import tpu_sc as plsc

NC = 2
NS = 16
L = 16
NW = NC * NS

BATCH = 16384
EMB = 64
ENT = 1000000
RELPAD = 1024
B_W = BATCH // NW

W = 512
RANGE = 61 * W
NCHUNK = 62
LAST_C0 = ENT - EMB
CLAMP = (ENT - W) // W * W
MCAP = 3072
CCAP = 256
GROWS = 4 * BATCH + NW * CCAP


def _rsqrt16(x):
    i = plsc.bitcast(x, jnp.int32)
    i = jnp.int32(0x5F3759DF) - (i >> 1)
    y = plsc.bitcast(i, jnp.float32)
    half = x * jnp.float32(0.5)
    for _ in range(3):
        y = y * (jnp.float32(1.5) - half * y * y)
    return y


def _scan_body(ph_h, pt_h, nh_h, nt_h, entT_h, gath_out,
               piece_v, me_v, md_v, cb0, cb1, pbuf, cmc_v, cmd_v,
               stage_v, sidx_v, sem_c, sem_a):
    wid = lax.axis_index("s") * NC + lax.axis_index("c")
    iota = lax.iota(jnp.int32, L)
    lo = wid * RANGE
    hi = jnp.where(wid == NW - 1, ENT, lo + RANGE)
    dump = 4 * BATCH + wid * CCAP

    cnt = jnp.int32(0)
    for p, side in enumerate((ph_h, pt_h, nh_h, nt_h)):
        for q in range(4):
            pltpu.sync_copy(side.at[pl.ds(q * 4096, 4096)], piece_v)
            dbase = p * BATCH + q * 4096

            def fbody(j, cnt):
                e16 = piece_v[pl.ds(j * L, L)]
                m = (e16 >= lo) & (e16 < hi)
                plsc.store_compressed(me_v.at[pl.ds(cnt, L)], e16, mask=m)
                d16 = dbase + j * L + iota
                plsc.store_compressed(md_v.at[pl.ds(cnt, L)], d16, mask=m)
                return cnt + plsc.all_reduce_population_count(m)[0]

            cnt = lax.fori_loop(0, 4096 // L, fbody, cnt)

    for j in range(CCAP // L):
        cmc_v[pl.ds(j * L, L)] = jnp.zeros((L,), jnp.int32)

    def process(cb, c0, cw):
        def mbody(j, mcnt):
            gi = j * L + iota
            e16 = me_v[pl.ds(j * L, L)]
            m = (e16 >= c0) & (e16 < c0 + cw) & (gi < cnt)
            plsc.store_compressed(cmc_v.at[pl.ds(mcnt, L)], e16 - c0, mask=m)
            d16 = md_v[pl.ds(j * L, L)]
            plsc.store_compressed(cmd_v.at[pl.ds(mcnt, L)], d16, mask=m)
            return mcnt + plsc.all_reduce_population_count(m)[0]

        mcnt = lax.fori_loop(0, MCAP // L, mbody, jnp.int32(0))

        def ebody(g, _):
            slot = g * L + iota
            cols = cmc_v[pl.ds(g * L, L)]
            valid = slot < mcnt
            for dd in range(EMB):
                dv = jnp.full((L,), dd, jnp.int32)
                vals = plsc.load_gather(cb, [dv, cols], mask=valid)
                plsc.store_scatter(stage_v, [slot, dv], vals, mask=valid)
            return 0

        lax.fori_loop(0, (mcnt + L - 1) >> 4, ebody, 0)

        def sbody(j, _):
            slot = j * L + iota
            dv = jnp.where(slot < mcnt, cmd_v[pl.ds(j * L, L)],
                           dump + slot)
            sidx_v[j >> 3, pl.ds((j & 7) * L, L)] = dv
            return 0

        lax.fori_loop(0, 2 * 128 // L, sbody, 0)

        pltpu.async_copy(stage_v.at[pl.ds(0, 128)],
                         gath_out.at[sidx_v.at[0]], sem_a).wait()

        @pl.when(mcnt > 128)
        def _():
            pltpu.async_copy(stage_v.at[pl.ds(128, 128)],
                             gath_out.at[sidx_v.at[1]], sem_a).wait()

    def fire(cidx, cb):
        c0 = pl.multiple_of(jnp.minimum(lo + cidx * W, CLAMP), W)
        return pltpu.async_copy(entT_h.at[:, pl.ds(c0, W)], cb, sem_c)

    fire(jnp.int32(0), cb0)
    fire(jnp.int32(1), cb1)

    def chunk_pair(i, _):
        for sub, cb in ((0, cb0), (1, cb1)):
            c = 2 * i + sub
            pltpu.make_async_copy(entT_h.at[:, pl.ds(0, W)], cb, sem_c).wait()
            process(cb, lo + c * W, W)
            fire(c + 2, cb)
        return 0

    lax.fori_loop(0, NCHUNK // 2, chunk_pair, 0)
    pltpu.make_async_copy(entT_h.at[:, pl.ds(0, W)], cb0, sem_c).wait()
    pltpu.make_async_copy(entT_h.at[:, pl.ds(0, W)], cb1, sem_c).wait()

    @pl.when(wid == NW - 1)
    def _():
        pltpu.sync_copy(entT_h.at[:, pl.ds(LAST_C0, EMB)], pbuf)
        process(pbuf, jnp.int32(LAST_C0), EMB)


def _compute_body(gath_h, pr_h, nr_h, relT_h, pos_out, neg_out,
                  relv, hbuf, tbuf, ridx_v, res_v, sem):
    wid = lax.axis_index("s") * NC + lax.axis_index("c")
    iota = lax.iota(jnp.int32, L)
    pltpu.sync_copy(relT_h, relv)

    for side, (r_h, out_hbm) in enumerate(((pr_h, pos_out), (nr_h, neg_out))):
        hbase = (2 * side) * BATCH + wid * B_W
        tbase = (2 * side + 1) * BATCH + wid * B_W
        pltpu.sync_copy(r_h.at[pl.ds(wid * B_W, B_W)], ridx_v)
        for sub in range(B_W // 128):
            pltpu.sync_copy(gath_h.at[pl.ds(hbase + sub * 128, 128)], hbuf)
            pltpu.sync_copy(gath_h.at[pl.ds(tbase + sub * 128, 128)], tbuf)

            def block(b, _):
                lb = b * L + iota
                r16 = ridx_v[pl.ds(sub * 128 + b * L, L)]
                zero = jnp.zeros((L,), jnp.float32)

                def norm_body(g, carry):
                    ah, at = carry
                    for k in range(L):
                        dv = jnp.full((L,), g * L + k, jnp.int32)
                        hc = plsc.load_gather(hbuf, [lb, dv])
                        tc = plsc.load_gather(tbuf, [lb, dv])
                        ah = ah + hc * hc
                        at = at + tc * tc
                    return ah, at

                nh2, nt2 = lax.fori_loop(0, EMB // L, norm_body, (zero, zero))
                invh = _rsqrt16(nh2)
                invt = _rsqrt16(nt2)

                def dist_body(g, acc):
                    for k in range(L):
                        dv = jnp.full((L,), g * L + k, jnp.int32)
                        hc = plsc.load_gather(hbuf, [lb, dv])
                        tc = plsc.load_gather(tbuf, [lb, dv])
                        rc = plsc.load_gather(relv, [dv, r16])
                        acc = acc + jnp.abs(hc * invh + rc - tc * invt)
                    return acc

                acc = lax.fori_loop(0, EMB // L, dist_body, zero)
                res_v[pl.ds(sub * 128 + b * L, L)] = acc
                return 0

            lax.fori_loop(0, 128 // L, block, 0)
        pltpu.sync_copy(res_v, out_hbm.at[pl.ds(wid * B_W, B_W)])


@jax.jit
def kernel(pos_h, pos_t, pos_r, neg_h, neg_t, neg_r, ent_emb, rel_emb):
    mesh = plsc.VectorSubcoreMesh(core_axis_name="c", subcore_axis_name="s",
                                  num_cores=NC, num_subcores=NS)
    params = pltpu.CompilerParams(needs_layout_passes=False)

    scan = pl.kernel(
        _scan_body,
        out_type=jax.ShapeDtypeStruct((GROWS, 2 * EMB), jnp.float32),
        mesh=mesh,
        scratch_types=[
            pltpu.VMEM((4096,), jnp.int32),
            pltpu.VMEM((MCAP,), jnp.int32),
            pltpu.VMEM((MCAP,), jnp.int32),
            pltpu.VMEM((EMB, W), jnp.float32),
            pltpu.VMEM((EMB, W), jnp.float32),
            pltpu.VMEM((EMB, EMB), jnp.float32),
            pltpu.VMEM((CCAP,), jnp.int32),
            pltpu.VMEM((CCAP,), jnp.int32),
            pltpu.VMEM((CCAP, 2 * EMB), jnp.float32),
            pltpu.VMEM((2, 128), jnp.int32),
            pltpu.SemaphoreType.DMA,
            pltpu.SemaphoreType.DMA,
        ],
        compiler_params=params,
    )
    gath = scan(pos_h, pos_t, neg_h, neg_t, ent_emb.T)

    relT = jnp.pad(rel_emb.T, ((0, 0), (0, RELPAD - rel_emb.shape[0])))
    compute = pl.kernel(
        _compute_body,
        out_type=(jax.ShapeDtypeStruct((BATCH,), jnp.float32),
                  jax.ShapeDtypeStruct((BATCH,), jnp.float32)),
        mesh=mesh,
        scratch_types=[
            pltpu.VMEM((EMB, RELPAD), jnp.float32),
            pltpu.VMEM((128, 2 * EMB), jnp.float32),
            pltpu.VMEM((128, 2 * EMB), jnp.float32),
            pltpu.VMEM((B_W,), jnp.int32),
            pltpu.VMEM((B_W,), jnp.float32),
            pltpu.SemaphoreType.DMA,
        ],
        compiler_params=params,
    )
    return compute(gath, pos_r, neg_r, relT)


# speedup vs baseline: 1.6973x; 1.1217x over previous
import jax
import jax.numpy as jnp
from jax import lax
from jax.experimental import pallas as pl
from jax.experimental.pallas import tpu as pltpu
from jax.experimental.pallas import tpu_sc as plsc

NC = 2
NS = 16
L = 16
NW = NC * NS

BATCH = 16384
EMB = 64
ENT = 1000000
RELPAD = 1024
B_W = BATCH // NW

W = 512
RANGE = 61 * W
NCHUNK = 62
LAST_C0 = ENT - EMB
CLAMP = (ENT - W) // W * W
MCAP = 3072
CCAP = 256
GROWS = 4 * BATCH + NW * CCAP


def _rsqrt16(x):
    i = plsc.bitcast(x, jnp.int32)
    i = jnp.int32(0x5F3759DF) - (i >> 1)
    y = plsc.bitcast(i, jnp.float32)
    half = x * jnp.float32(0.5)
    for _ in range(3):
        y = y * (jnp.float32(1.5) - half * y * y)
    return y


def _scan_body(ph_h, pt_h, nh_h, nt_h, entT_h, gath_out,
               piece_v, me_v, md_v, cb0, cb1, pbuf, cmc_v, cmd_v,
               stage_v, sidx_v, sem_c, sem_a):
    wid = lax.axis_index("s") * NC + lax.axis_index("c")
    iota = lax.iota(jnp.int32, L)
    lo = wid * RANGE
    hi = jnp.where(wid == NW - 1, ENT, lo + RANGE)
    dump = 4 * BATCH + wid * CCAP

    cnt = jnp.int32(0)
    for p, side in enumerate((ph_h, pt_h, nh_h, nt_h)):
        for q in range(4):
            pltpu.sync_copy(side.at[pl.ds(q * 4096, 4096)], piece_v)
            dbase = p * BATCH + q * 4096

            def fbody(j, cnt):
                e16 = piece_v[pl.ds(j * L, L)]
                m = (e16 >= lo) & (e16 < hi)
                plsc.store_compressed(me_v.at[pl.ds(cnt, L)], e16, mask=m)
                d16 = dbase + j * L + iota
                plsc.store_compressed(md_v.at[pl.ds(cnt, L)], d16, mask=m)
                return cnt + plsc.all_reduce_population_count(m)[0]

            cnt = lax.fori_loop(0, 4096 // L, fbody, cnt)

    for j in range(CCAP // L):
        cmc_v[pl.ds(j * L, L)] = jnp.zeros((L,), jnp.int32)

    def process(cb, c0, cw):
        def mbody(j, mcnt):
            gi = j * L + iota
            e16 = me_v[pl.ds(j * L, L)]
            m = (e16 >= c0) & (e16 < c0 + cw) & (gi < cnt)
            plsc.store_compressed(cmc_v.at[pl.ds(mcnt, L)], e16 - c0, mask=m)
            d16 = md_v[pl.ds(j * L, L)]
            plsc.store_compressed(cmd_v.at[pl.ds(mcnt, L)], d16, mask=m)
            return mcnt + plsc.all_reduce_population_count(m)[0]

        mcnt = lax.fori_loop(0, (cnt + L - 1) >> 4, mbody, jnp.int32(0))

        def ebody(g, _):
            slot = g * L + iota
            cols = cmc_v[pl.ds(g * L, L)]
            valid = slot < mcnt
            for dd in range(EMB):
                dv = jnp.full((L,), dd, jnp.int32)
                vals = plsc.load_gather(cb, [dv, cols], mask=valid)
                plsc.store_scatter(stage_v, [slot, dv], vals, mask=valid)
            return 0

        lax.fori_loop(0, (mcnt + L - 1) >> 4, ebody, 0)

        def sbody(j, _):
            slot = j * L + iota
            dv = jnp.where(slot < mcnt, cmd_v[pl.ds(j * L, L)],
                           dump + slot)
            sidx_v[j >> 3, pl.ds((j & 7) * L, L)] = dv
            return 0

        lax.fori_loop(0, 2 * 128 // L, sbody, 0)

        pltpu.async_copy(stage_v.at[pl.ds(0, 128)],
                         gath_out.at[sidx_v.at[0]], sem_a).wait()

        @pl.when(mcnt > 128)
        def _():
            pltpu.async_copy(stage_v.at[pl.ds(128, 128)],
                             gath_out.at[sidx_v.at[1]], sem_a).wait()

    def fire(cidx, cb):
        c0 = pl.multiple_of(jnp.minimum(lo + cidx * W, CLAMP), W)
        return pltpu.async_copy(entT_h.at[:, pl.ds(c0, W)], cb, sem_c)

    fire(jnp.int32(0), cb0)
    fire(jnp.int32(1), cb1)

    def chunk_pair(i, _):
        for sub, cb in ((0, cb0), (1, cb1)):
            c = 2 * i + sub
            pltpu.make_async_copy(entT_h.at[:, pl.ds(0, W)], cb, sem_c).wait()
            process(cb, lo + c * W, W)
            fire(c + 2, cb)
        return 0

    lax.fori_loop(0, NCHUNK // 2, chunk_pair, 0)
    pltpu.make_async_copy(entT_h.at[:, pl.ds(0, W)], cb0, sem_c).wait()
    pltpu.make_async_copy(entT_h.at[:, pl.ds(0, W)], cb1, sem_c).wait()

    @pl.when(wid == NW - 1)
    def _():
        pltpu.sync_copy(entT_h.at[:, pl.ds(LAST_C0, EMB)], pbuf)
        process(pbuf, jnp.int32(LAST_C0), EMB)


def _compute_body(gath_h, pr_h, nr_h, relT_h, pos_out, neg_out,
                  relv, hb0, hb1, tb0, tb1, ridx_v, res_v, sem):
    wid = lax.axis_index("s") * NC + lax.axis_index("c")
    iota = lax.iota(jnp.int32, L)
    pltpu.sync_copy(relT_h, relv)
    SUB = 64
    NSUB = B_W // SUB
    hbufs, tbufs = (hb0, hb1), (tb0, tb1)

    for side, (r_h, out_hbm) in enumerate(((pr_h, pos_out), (nr_h, neg_out))):
        hbase = (2 * side) * BATCH + wid * B_W
        tbase = (2 * side + 1) * BATCH + wid * B_W
        pltpu.sync_copy(r_h.at[pl.ds(wid * B_W, B_W)], ridx_v)

        def fire(sub):
            p = sub & 1
            return (pltpu.async_copy(
                        gath_h.at[pl.ds(hbase + sub * SUB, SUB)],
                        hbufs[p], sem),
                    pltpu.async_copy(
                        gath_h.at[pl.ds(tbase + sub * SUB, SUB)],
                        tbufs[p], sem))

        descs = fire(0)
        for sub in range(NSUB):
            for d in descs:
                d.wait()
            if sub + 1 < NSUB:
                descs = fire(sub + 1)
            hbuf, tbuf = hbufs[sub & 1], tbufs[sub & 1]

            def block(b, _):
                lb = b * L + iota
                r16 = ridx_v[pl.ds(sub * SUB + b * L, L)]
                zero = jnp.zeros((L,), jnp.float32)

                def norm_body(g, carry):
                    ah, at = carry
                    for k in range(L):
                        dv = jnp.full((L,), g * L + k, jnp.int32)
                        hc = plsc.load_gather(hbuf, [lb, dv])
                        tc = plsc.load_gather(tbuf, [lb, dv])
                        ah = ah + hc * hc
                        at = at + tc * tc
                    return ah, at

                nh2, nt2 = lax.fori_loop(0, EMB // L, norm_body, (zero, zero))
                invh = _rsqrt16(nh2)
                invt = _rsqrt16(nt2)

                def dist_body(g, acc):
                    for k in range(L):
                        dv = jnp.full((L,), g * L + k, jnp.int32)
                        hc = plsc.load_gather(hbuf, [lb, dv])
                        tc = plsc.load_gather(tbuf, [lb, dv])
                        rc = plsc.load_gather(relv, [dv, r16])
                        acc = acc + jnp.abs(hc * invh + rc - tc * invt)
                    return acc

                acc = lax.fori_loop(0, EMB // L, dist_body, zero)
                res_v[pl.ds(sub * SUB + b * L, L)] = acc
                return 0

            lax.fori_loop(0, SUB // L, block, 0)
        pltpu.sync_copy(res_v, out_hbm.at[pl.ds(wid * B_W, B_W)])


@jax.jit
def kernel(pos_h, pos_t, pos_r, neg_h, neg_t, neg_r, ent_emb, rel_emb):
    mesh = plsc.VectorSubcoreMesh(core_axis_name="c", subcore_axis_name="s",
                                  num_cores=NC, num_subcores=NS)
    params = pltpu.CompilerParams(needs_layout_passes=False)

    scan = pl.kernel(
        _scan_body,
        out_type=jax.ShapeDtypeStruct((GROWS, 2 * EMB), jnp.float32),
        mesh=mesh,
        scratch_types=[
            pltpu.VMEM((4096,), jnp.int32),
            pltpu.VMEM((MCAP,), jnp.int32),
            pltpu.VMEM((MCAP,), jnp.int32),
            pltpu.VMEM((EMB, W), jnp.float32),
            pltpu.VMEM((EMB, W), jnp.float32),
            pltpu.VMEM((EMB, EMB), jnp.float32),
            pltpu.VMEM((CCAP,), jnp.int32),
            pltpu.VMEM((CCAP,), jnp.int32),
            pltpu.VMEM((CCAP, 2 * EMB), jnp.float32),
            pltpu.VMEM((2, 128), jnp.int32),
            pltpu.SemaphoreType.DMA,
            pltpu.SemaphoreType.DMA,
        ],
        compiler_params=params,
    )
    gath = scan(pos_h, pos_t, neg_h, neg_t, ent_emb.T)

    relT = jnp.pad(rel_emb.T, ((0, 0), (0, RELPAD - rel_emb.shape[0])))
    compute = pl.kernel(
        _compute_body,
        out_type=(jax.ShapeDtypeStruct((BATCH,), jnp.float32),
                  jax.ShapeDtypeStruct((BATCH,), jnp.float32)),
        mesh=mesh,
        scratch_types=[
            pltpu.VMEM((EMB, RELPAD), jnp.float32),
            pltpu.VMEM((64, 2 * EMB), jnp.float32),
            pltpu.VMEM((64, 2 * EMB), jnp.float32),
            pltpu.VMEM((64, 2 * EMB), jnp.float32),
            pltpu.VMEM((64, 2 * EMB), jnp.float32),
            pltpu.VMEM((B_W,), jnp.int32),
            pltpu.VMEM((B_W,), jnp.float32),
            pltpu.SemaphoreType.DMA,
        ],
        compiler_params=params,
    )
    return compute(gath, pos_r, neg_r, relT)
